# Initial kernel scaffold; baseline (speedup 1.0000x reference)
import jax
import jax.numpy as jnp
from jax.experimental import pallas as pl


def kernel(x, token_table, pos_table):
    raise NotImplementedError("write your pallas kernel here")


# speedup vs baseline: 2.2135x; 2.2135x over previous
import functools

import jax
import jax.numpy as jnp
from jax import lax
from jax.experimental import pallas as pl
from jax.experimental.pallas import tpu as pltpu
from jax.experimental.pallas import tpu_sc as plsc

VOCAB = 1000000
MAXLEN = 200
EMBED_DIM = 64
BATCH = 4096
BL = BATCH * MAXLEN
NC, NS, LANES = 2, 16, 16
NW = NC * NS
PER_W = BL // NW
G = 128
C = 512
NG = C // G
NCHUNK = PER_W // C


def _body(xf, tok_tab, pos_tab, out, idx_v, pidx_v, tok_v, pos_v, sem_t,
          sem_p):
  wid = lax.axis_index("s") * NC + lax.axis_index("c")
  base = wid * PER_W

  @pl.loop(0, NCHUNK)
  def _chunk(ci):
    off = base + ci * C
    pltpu.sync_copy(xf.at[pl.ds(off, C)], idx_v)

    tok_dmas = []
    for g in range(NG):
      tok_dmas.append(
          pltpu.async_copy(tok_tab.at[idx_v.at[pl.ds(g * G, G)]],
                           tok_v.at[pl.ds(g * G, G)], sem_t))

    for i in range(C // LANES):
      s = pl.ds(i * LANES, LANES)
      xi = idx_v[s]
      l = (off + (i * LANES + lax.iota(jnp.int32, 16))) % MAXLEN
      pidx_v[s] = jnp.where(xi > 0, l + 1, 0)

    pos_dmas = []
    for g in range(NG):
      pos_dmas.append(
          pltpu.async_copy(pos_tab.at[pidx_v.at[pl.ds(g * G, G)]],
                           pos_v.at[pl.ds(g * G, G)], sem_p))
    for d in tok_dmas:
      d.wait()
    for d in pos_dmas:
      d.wait()

    @pl.loop(0, C)
    def _add(j):
      for q in range(EMBED_DIM // LANES):
        s = pl.ds(q * LANES, LANES)
        tok_v[j, s] = tok_v[j, s] + pos_v[j, s]

    pltpu.sync_copy(tok_v, out.at[pl.ds(off, C)])


@functools.partial(jax.jit, donate_argnums=())
def kernel(x, token_table, pos_table):
  mesh = plsc.VectorSubcoreMesh(core_axis_name="c", subcore_axis_name="s")
  run = pl.kernel(
      _body,
      out_type=jax.ShapeDtypeStruct((BL, EMBED_DIM), jnp.float32),
      mesh=mesh,
      scratch_types=[
          pltpu.VMEM((C,), jnp.int32),
          pltpu.VMEM((C,), jnp.int32),
          pltpu.VMEM((C, EMBED_DIM), jnp.float32),
          pltpu.VMEM((C, EMBED_DIM), jnp.float32),
          pltpu.SemaphoreType.DMA,
          pltpu.SemaphoreType.DMA,
      ],
      compiler_params=pltpu.CompilerParams(use_tc_tiling_on_sc=False),
  )
  out = run(x.reshape(BL), token_table, pos_table)
  return out.reshape(BATCH, MAXLEN, EMBED_DIM)
